# baseline (speedup 1.0000x reference)
import functools

import jax
import jax.numpy as jnp
from jax import lax
from jax.experimental import pallas as pl
from jax.experimental.pallas import tpu as pltpu
from jax.experimental.pallas import tpu_sc as plsc

QUARTER = 192
SEQ = 81
HID = 4 * QUARTER
ENC = SEQ * HID
NC = 2
NS = 16
NW = NC * NS


def _sc_kernel(nbatch, row_hbm, col_hbm, box_hbm, pos_hbm, out_hbm,
               enc_v, build_sem, out_sem):
    copies = []
    for p in range(SEQ):
        r, c = p // 9, p % 9
        b = (r // 3) * 3 + (c // 3)
        base = p * HID
        for q, (tab, idx) in enumerate(
            ((row_hbm, r), (col_hbm, c), (box_hbm, b), (pos_hbm, p))
        ):
            copies.append(pltpu.make_async_copy(
                tab.at[pl.ds(idx * QUARTER, QUARTER)],
                enc_v.at[pl.ds(base + q * QUARTER, QUARTER)],
                build_sem))
    for cp in copies:
        cp.start()
    for cp in copies:
        cp.wait()

    wid = lax.axis_index("s") * NC + lax.axis_index("c")
    per_w = nbatch // NW
    base = wid * (per_w * ENC)
    outs = [
        pltpu.make_async_copy(
            enc_v, out_hbm.at[pl.ds(base + j * ENC, ENC)], out_sem)
        for j in range(per_w)
    ]
    for cp in outs:
        cp.start()
    for cp in outs:
        cp.wait()


@functools.partial(jax.jit, static_argnames=("batch",))
def _run(row_embed, col_embed, box_embed, pos_embed, batch):
    assert batch % NW == 0
    mesh = plsc.VectorSubcoreMesh(core_axis_name="c", subcore_axis_name="s")
    kern = pl.kernel(
        functools.partial(_sc_kernel, batch),
        out_type=jax.ShapeDtypeStruct((batch * ENC,), jnp.float32),
        mesh=mesh,
        scratch_types=[
            pltpu.VMEM((ENC,), jnp.float32),
            pltpu.SemaphoreType.DMA,
            pltpu.SemaphoreType.DMA,
        ],
    )
    out = kern(
        row_embed.reshape(-1),
        col_embed.reshape(-1),
        box_embed.reshape(-1),
        pos_embed.reshape(-1),
    )
    return out.reshape(batch, SEQ, HID)


def kernel(x, row_embed, col_embed, box_embed, pos_embed):
    batch = x.shape[0]
    return _run(row_embed, col_embed, box_embed, pos_embed, batch)


# speedup vs baseline: 1.8022x; 1.8022x over previous
import functools

import jax
import jax.numpy as jnp
from jax import lax
from jax.experimental import pallas as pl
from jax.experimental.pallas import tpu as pltpu
from jax.experimental.pallas import tpu_sc as plsc

QUARTER = 192
SEQ = 81
HID = 4 * QUARTER
NC = 2
NS = 16
NW = NC * NS


def _enc_from_tables(row, col, box, pos):
    row81 = jnp.broadcast_to(row[:, None, :], (9, 9, QUARTER)).reshape(SEQ, QUARTER)
    col81 = jnp.broadcast_to(col[None, :, :], (9, 9, QUARTER)).reshape(SEQ, QUARTER)
    boxr = box.reshape(3, 3, QUARTER)
    box81 = jnp.broadcast_to(
        boxr[:, None, :, None, :], (3, 3, 3, 3, QUARTER)
    ).reshape(SEQ, QUARTER)
    return jnp.concatenate([row81, col81, box81, pos], axis=-1)


def _enc_kernel(row_ref, col_ref, box_ref, pos_ref, enc_ref):
    enc_ref[:] = _enc_from_tables(
        row_ref[:], col_ref[:], box_ref[:], pos_ref[:]
    )


def _sc_bcast(nbatch, enc_hbm, out_hbm, enc_v, in_sem, out_sem):
    pltpu.make_async_copy(enc_hbm, enc_v, in_sem).start()
    pltpu.make_async_copy(enc_hbm, enc_v, in_sem).wait()
    wid = lax.axis_index("s") * NC + lax.axis_index("c")
    per_w = nbatch // NW
    base = wid * per_w
    outs = [
        pltpu.make_async_copy(enc_v, out_hbm.at[base + j], out_sem)
        for j in range(per_w)
    ]
    for cp in outs:
        cp.start()
    for cp in outs:
        cp.wait()


@functools.partial(jax.jit, static_argnames=("batch",))
def _run(row_embed, col_embed, box_embed, pos_embed, batch):
    assert batch % NW == 0
    enc = pl.pallas_call(
        _enc_kernel,
        out_shape=jax.ShapeDtypeStruct((SEQ, HID), jnp.float32),
    )(row_embed, col_embed, box_embed, pos_embed)

    mesh = plsc.VectorSubcoreMesh(core_axis_name="c", subcore_axis_name="s")
    kern = pl.kernel(
        functools.partial(_sc_bcast, batch),
        out_type=jax.ShapeDtypeStruct((batch, SEQ, HID), jnp.float32),
        mesh=mesh,
        scratch_types=[
            pltpu.VMEM((SEQ, HID), jnp.float32),
            pltpu.SemaphoreType.DMA,
            pltpu.SemaphoreType.DMA,
        ],
        compiler_params=pltpu.CompilerParams(use_tc_tiling_on_sc=True),
    )
    return kern(enc)


def kernel(x, row_embed, col_embed, box_embed, pos_embed):
    batch = x.shape[0]
    return _run(row_embed, col_embed, box_embed, pos_embed, batch)


# speedup vs baseline: 6.4496x; 3.5788x over previous
import functools

import jax
import jax.numpy as jnp
from jax.experimental import pallas as pl
from jax.experimental.pallas import tpu as pltpu

QUARTER = 192
SEQ = 81
HID = 4 * QUARTER
BS = 3


def _one_hot_rows(idx, n, table):
    j = jax.lax.broadcasted_iota(jnp.int32, (BS, n), 1)
    oh = (j == idx[:, None]).astype(jnp.float32)
    return jax.lax.dot_general(
        oh, table, (((1,), (0,)), ((), ())),
        preferred_element_type=jnp.float32)


def _bcast_kernel(batch, row_ref, col_ref, box_ref, pos_ref, out_ref):
    i = pl.program_id(0)
    p = i * BS + jax.lax.broadcasted_iota(jnp.int32, (BS,), 0)
    r, c = p // 9, p % 9
    b = (r // 3) * 3 + c // 3
    rows = jnp.concatenate([
        _one_hot_rows(r, 9, row_ref[:]),
        _one_hot_rows(c, 9, col_ref[:]),
        _one_hot_rows(b, 9, box_ref[:]),
        _one_hot_rows(p, SEQ, pos_ref[:]),
    ], axis=-1)
    out_ref[:] = jnp.broadcast_to(rows[:, None, :], (BS, batch, HID))


@functools.partial(jax.jit, static_argnames=("batch",))
def _run(row_embed, col_embed, box_embed, pos_embed, batch):
    grid = (SEQ // BS,)
    out = pl.pallas_call(
        functools.partial(_bcast_kernel, batch),
        grid=grid,
        in_specs=[
            pl.BlockSpec((9, QUARTER), lambda i: (0, 0)),
            pl.BlockSpec((9, QUARTER), lambda i: (0, 0)),
            pl.BlockSpec((9, QUARTER), lambda i: (0, 0)),
            pl.BlockSpec((SEQ, QUARTER), lambda i: (0, 0)),
        ],
        out_specs=pl.BlockSpec((BS, batch, HID), lambda i: (i, 0, 0)),
        out_shape=jax.ShapeDtypeStruct((SEQ, batch, HID), jnp.float32),
        compiler_params=pltpu.CompilerParams(
            dimension_semantics=("parallel",),
        ),
    )(row_embed, col_embed, box_embed, pos_embed)
    return jnp.transpose(out, (1, 0, 2))


def kernel(x, row_embed, col_embed, box_embed, pos_embed):
    batch = x.shape[0]
    return _run(row_embed, col_embed, box_embed, pos_embed, batch)
